# baseline (speedup 1.0000x reference)
import jax
import jax.numpy as jnp
from jax.experimental import pallas as pl
from jax.experimental.pallas import tpu as pltpu


_CB = 32


def _mul_kernel(um_ref, x_ref, o_ref):
    c0 = pl.program_id(1) * _CB
    for i in range(_CB):
        s = um_ref[0, c0 + i]
        o_ref[0, i] = x_ref[0, i] * s


def kernel(input, unit_mapping):
    B, C, H, W = input.shape
    grid = (B, C // _CB)
    out = pl.pallas_call(
        _mul_kernel,
        grid=grid,
        in_specs=[
            pl.BlockSpec(memory_space=pltpu.SMEM),
            pl.BlockSpec((1, _CB, H, W), lambda b, c: (b, c, 0, 0)),
        ],
        out_specs=pl.BlockSpec((1, _CB, H, W), lambda b, c: (b, c, 0, 0)),
        out_shape=jax.ShapeDtypeStruct((B, C, H, W), input.dtype),
        compiler_params=pltpu.CompilerParams(
            dimension_semantics=("parallel", "parallel")),
    )(unit_mapping, input)
    return out


# speedup vs baseline: 1.0048x; 1.0048x over previous
import jax
import jax.numpy as jnp
from jax.experimental import pallas as pl
from jax.experimental.pallas import tpu as pltpu


_CB = 48


def _mul_kernel(um_ref, x_ref, o_ref):
    c0 = pl.program_id(1) * _CB
    for i in range(_CB):
        s = um_ref[0, c0 + i]
        o_ref[0, i] = x_ref[0, i] * s


def kernel(input, unit_mapping):
    B, C, H, W = input.shape
    grid = (B, C // _CB)
    out = pl.pallas_call(
        _mul_kernel,
        grid=grid,
        in_specs=[
            pl.BlockSpec(memory_space=pltpu.SMEM),
            pl.BlockSpec((1, _CB, H, W), lambda b, c: (b, c, 0, 0)),
        ],
        out_specs=pl.BlockSpec((1, _CB, H, W), lambda b, c: (b, c, 0, 0)),
        out_shape=jax.ShapeDtypeStruct((B, C, H, W), input.dtype),
        compiler_params=pltpu.CompilerParams(
            dimension_semantics=("parallel", "parallel")),
    )(unit_mapping, input)
    return out


# speedup vs baseline: 1.0048x; 1.0000x over previous
import jax
import jax.numpy as jnp
from jax.experimental import pallas as pl
from jax.experimental.pallas import tpu as pltpu


_CB = 48


def _mul_kernel(um_ref, x_ref, o_ref):
    c0 = pl.program_id(1) * _CB
    for i in range(_CB):
        s = um_ref[0, c0 + i]
        o_ref[0, i] = x_ref[0, i] * s


def kernel(input, unit_mapping):
    B, C, H, W = input.shape
    grid = (B, C // _CB)
    out = pl.pallas_call(
        _mul_kernel,
        grid=grid,
        in_specs=[
            pl.BlockSpec(memory_space=pltpu.SMEM),
            pl.BlockSpec((1, _CB, H, W), lambda b, c: (b, c, 0, 0)),
        ],
        out_specs=pl.BlockSpec((1, _CB, H, W), lambda b, c: (b, c, 0, 0)),
        out_shape=jax.ShapeDtypeStruct((B, C, H, W), input.dtype),
        compiler_params=pltpu.CompilerParams(
            dimension_semantics=("arbitrary", "arbitrary")),
    )(unit_mapping, input)
    return out
